# baseline (speedup 1.0000x reference)
import jax
import jax.numpy as jnp
from jax.experimental import pallas as pl


_BS = 1024
_BD = 512


def _add_body(x_ref, pos_ref, o_ref):
    o_ref[...] = x_ref[...] + pos_ref[...][:, None, :]


def kernel(x, pos_table):
    s, batch, d = x.shape
    return pl.pallas_call(
        _add_body,
        grid=(s // _BS, d // _BD),
        in_specs=[
            pl.BlockSpec((_BS, batch, _BD), lambda i, j: (i, 0, j)),
            pl.BlockSpec((_BS, _BD), lambda i, j: (i, j)),
        ],
        out_specs=pl.BlockSpec((_BS, batch, _BD), lambda i, j: (i, 0, j)),
        out_shape=jax.ShapeDtypeStruct((s, batch, d), x.dtype),
    )(x, pos_table)
